# baseline (speedup 1.0000x reference)
import functools

import jax
import jax.numpy as jnp
from jax import lax
from jax.experimental import pallas as pl
from jax.experimental.pallas import tpu as pltpu
from jax.experimental.pallas import tpu_sc as plsc

B, L, D = 4, 2048, 512
K, C = 8192, 16
CP = 128
TOK = 256
KT = 1024
N_TILES = (B * L) // TOK

NC, NS = 2, 16
NW = NC * NS
CAND = 2 * B * L
CAND_W = CAND // NW
GCHUNK = 128


def _scan_body(x_ref, w_ref, cbt_ref, t_ref, i1_ref, i2_ref):
    x = x_ref[...]
    w = w_ref[...]
    cbt = cbt_ref[...]
    t = lax.dot_general(x, w, (((1,), (1,)), ((), ())),
                        preferred_element_type=jnp.float32)
    t_ref[...] = t
    c2 = jnp.sum(cbt * cbt, axis=0)
    t2 = t + t
    ii0 = lax.broadcasted_iota(jnp.int32, (TOK, KT), 1)

    inf = jnp.float32(jnp.inf)
    m1g = jnp.full((TOK,), inf, jnp.float32)
    i1g = jnp.zeros((TOK,), jnp.int32)
    m2g = jnp.full((TOK,), inf, jnp.float32)
    i2g = jnp.zeros((TOK,), jnp.int32)
    for j in range(K // KT):
        cbj = cbt[:, j * KT:(j + 1) * KT]
        s2 = lax.dot_general(t2, cbj, (((1,), (0,)), ((), ())),
                            precision=lax.Precision.HIGHEST,
                            preferred_element_type=jnp.float32)
        v = c2[j * KT:(j + 1) * KT][None, :] - s2
        m1 = jnp.min(v, axis=1)
        i1 = jnp.argmin(v, axis=1).astype(jnp.int32)
        vm = jnp.where(ii0 == i1[:, None], inf, v)
        m2 = jnp.min(vm, axis=1)
        i2 = jnp.argmin(vm, axis=1).astype(jnp.int32)
        i1 = i1 + (j * KT)
        i2 = i2 + (j * KT)
        for mv, iv in ((m1, i1), (m2, i2)):
            p1 = mv < m1g
            n1v = jnp.where(p1, mv, m1g)
            n1i = jnp.where(p1, iv, i1g)
            rv = jnp.where(p1, m1g, mv)
            ri = jnp.where(p1, i1g, iv)
            p2 = rv < m2g
            m2g = jnp.where(p2, rv, m2g)
            i2g = jnp.where(p2, ri, i2g)
            m1g, i1g = n1v, n1i
    i1_ref[0, 0, :] = i1g
    i2_ref[0, 0, :] = i2g


def _tc_scan(x, w, cbt):
    return pl.pallas_call(
        _scan_body,
        grid=(N_TILES,),
        in_specs=[
            pl.BlockSpec((TOK, D), lambda i: (i, 0)),
            pl.BlockSpec((C, D), lambda i: (0, 0)),
            pl.BlockSpec((C, K), lambda i: (0, 0)),
        ],
        out_specs=[
            pl.BlockSpec((TOK, C), lambda i: (i, 0)),
            pl.BlockSpec((1, 1, TOK), lambda i: (i, 0, 0)),
            pl.BlockSpec((1, 1, TOK), lambda i: (i, 0, 0)),
        ],
        out_shape=[
            jax.ShapeDtypeStruct((B * L, C), jnp.float32),
            jax.ShapeDtypeStruct((N_TILES, 1, TOK), jnp.int32),
            jax.ShapeDtypeStruct((N_TILES, 1, TOK), jnp.int32),
        ],
    )(x, w, cbt)


_GROWS = CAND_W // GCHUNK
_SC_GATHER = None


def _build_sc_gather():
    global _SC_GATHER
    if _SC_GATHER is None:
        mesh = plsc.VectorSubcoreMesh(core_axis_name="c", subcore_axis_name="s",
                                      num_cores=NC, num_subcores=NS)

        @functools.partial(
            pl.kernel,
            out_type=jax.ShapeDtypeStruct((CAND, CP), jnp.float32),
            mesh=mesh,
            scratch_types=[
                pltpu.VMEM((_GROWS, GCHUNK), jnp.int32),
                pltpu.VMEM((CAND_W, CP), jnp.float32),
                pltpu.SemaphoreType.DMA,
            ],
        )
        def _sc_gather(cb_hbm, idx_hbm, out_hbm, idx_v, rows_v, sem):
            wid = lax.axis_index("s") * NC + lax.axis_index("c")
            pltpu.sync_copy(idx_hbm.at[pl.ds(wid * _GROWS, _GROWS)], idx_v)
            for g in range(_GROWS):
                pltpu.async_copy(cb_hbm.at[idx_v.at[g]],
                                 rows_v.at[pl.ds(g * GCHUNK, GCHUNK)], sem).wait()
            pltpu.sync_copy(rows_v, out_hbm.at[pl.ds(wid * CAND_W, CAND_W)])

        _SC_GATHER = _sc_gather
    return _SC_GATHER


def _gather_rows(code_book_padded, idx):
    return _build_sc_gather()(code_book_padded,
                              idx.reshape(CAND // GCHUNK, GCHUNK))


def _sum_sq_tree(sq):
    while len(sq) > 1:
        h = len(sq) // 2
        sq = [sq[i] + sq[i + h] for i in range(h)]
    return sq[0]


def _dist_exact(t, rows):
    sq = []
    for c in range(C):
        dlt = t[:, c:c + 1] - rows[:, c:c + 1]
        sq.append(dlt * dlt)
    return jnp.sqrt(_sum_sq_tree(sq))


def _rerank_body(t_ref, r1_ref, r2_ref, i1_ref, i2_ref, out_ref):
    t = t_ref[...]
    d1 = _dist_exact(t, r1_ref[...])[:, 0]
    d2 = _dist_exact(t, r2_ref[...])[:, 0]
    i1 = i1_ref[0, :]
    i2 = i2_ref[0, :]
    pick2 = (d2 < d1) | ((d2 == d1) & (i2 < i1))
    out_ref[0, :] = jnp.where(pick2, i2, i1)


def _tc_rerank(t, rows1, rows2, i1, i2):
    return pl.pallas_call(
        _rerank_body,
        out_shape=jax.ShapeDtypeStruct((1, B * L), jnp.int32),
    )(t, rows1, rows2, i1[None, :], i2[None, :])


def kernel(input_values, proj_weight, code_book):
    x = input_values.reshape(B * L, D)
    cbp = jnp.pad(code_book, ((0, 0), (0, CP - C)))
    t, i1, i2 = _tc_scan(x, proj_weight, code_book.T)
    i1 = i1.reshape(B * L)
    i2 = i2.reshape(B * L)
    idx = jnp.concatenate([i1, i2])
    rows = _gather_rows(cbp, idx)
    labels = _tc_rerank(t, rows[:B * L], rows[B * L:], i1, i2)
    return labels.reshape(B, L)


# speedup vs baseline: 1.0628x; 1.0628x over previous
import functools

import jax
import jax.numpy as jnp
from jax import lax
from jax.experimental import pallas as pl
from jax.experimental.pallas import tpu as pltpu
from jax.experimental.pallas import tpu_sc as plsc

B, L, D = 4, 2048, 512
K, C = 8192, 16
CP = 128
TOK = 256
KT = 1024
N_TILES = (B * L) // TOK

NC, NS = 2, 16
NW = NC * NS
CAND = 2 * B * L
CAND_W = CAND // NW
GCHUNK = 128


def _scan_body(x_ref, w_ref, cbt_ref, t_ref, i1_ref, i2_ref):
    x = x_ref[...]
    w = w_ref[...]
    cbt = cbt_ref[...]
    t = lax.dot_general(x, w, (((1,), (1,)), ((), ())),
                        preferred_element_type=jnp.float32)
    t_ref[...] = t
    c2 = jnp.sum(cbt * cbt, axis=0)
    t2 = t + t
    ii0 = lax.broadcasted_iota(jnp.int32, (TOK, KT), 1)

    inf = jnp.float32(jnp.inf)
    m1g = jnp.full((TOK,), inf, jnp.float32)
    i1g = jnp.zeros((TOK,), jnp.int32)
    m2g = jnp.full((TOK,), inf, jnp.float32)
    i2g = jnp.zeros((TOK,), jnp.int32)
    for j in range(K // KT):
        cbj = cbt[:, j * KT:(j + 1) * KT]
        s2 = lax.dot_general(t2, cbj, (((1,), (0,)), ((), ())),
                            precision=lax.Precision.HIGHEST,
                            preferred_element_type=jnp.float32)
        v = c2[j * KT:(j + 1) * KT][None, :] - s2
        m1 = jnp.min(v, axis=1)
        i1 = jnp.min(jnp.where(v == m1[:, None], ii0, jnp.int32(K)), axis=1)
        vm = jnp.where(ii0 == i1[:, None], inf, v)
        m2 = jnp.min(vm, axis=1)
        i2 = jnp.min(jnp.where(vm == m2[:, None], ii0, jnp.int32(K)), axis=1)
        i1 = i1 + (j * KT)
        i2 = i2 + (j * KT)
        for mv, iv in ((m1, i1), (m2, i2)):
            p1 = mv < m1g
            n1v = jnp.where(p1, mv, m1g)
            n1i = jnp.where(p1, iv, i1g)
            rv = jnp.where(p1, m1g, mv)
            ri = jnp.where(p1, i1g, iv)
            p2 = rv < m2g
            m2g = jnp.where(p2, rv, m2g)
            i2g = jnp.where(p2, ri, i2g)
            m1g, i1g = n1v, n1i
    i1_ref[0, 0, :] = i1g
    i2_ref[0, 0, :] = i2g


def _tc_scan(x, w, cbt):
    return pl.pallas_call(
        _scan_body,
        grid=(N_TILES,),
        in_specs=[
            pl.BlockSpec((TOK, D), lambda i: (i, 0)),
            pl.BlockSpec((C, D), lambda i: (0, 0)),
            pl.BlockSpec((C, K), lambda i: (0, 0)),
        ],
        out_specs=[
            pl.BlockSpec((TOK, C), lambda i: (i, 0)),
            pl.BlockSpec((1, 1, TOK), lambda i: (i, 0, 0)),
            pl.BlockSpec((1, 1, TOK), lambda i: (i, 0, 0)),
        ],
        out_shape=[
            jax.ShapeDtypeStruct((B * L, C), jnp.float32),
            jax.ShapeDtypeStruct((N_TILES, 1, TOK), jnp.int32),
            jax.ShapeDtypeStruct((N_TILES, 1, TOK), jnp.int32),
        ],
    )(x, w, cbt)


_GROWS = CAND_W // GCHUNK
_SC_GATHER = None


def _build_sc_gather():
    global _SC_GATHER
    if _SC_GATHER is None:
        mesh = plsc.VectorSubcoreMesh(core_axis_name="c", subcore_axis_name="s",
                                      num_cores=NC, num_subcores=NS)

        @functools.partial(
            pl.kernel,
            out_type=jax.ShapeDtypeStruct((CAND, CP), jnp.float32),
            mesh=mesh,
            scratch_types=[
                pltpu.VMEM((_GROWS, GCHUNK), jnp.int32),
                pltpu.VMEM((CAND_W, CP), jnp.float32),
                pltpu.SemaphoreType.DMA,
            ],
        )
        def _sc_gather(cb_hbm, idx_hbm, out_hbm, idx_v, rows_v, sem):
            wid = lax.axis_index("s") * NC + lax.axis_index("c")
            pltpu.sync_copy(idx_hbm.at[pl.ds(wid * _GROWS, _GROWS)], idx_v)
            for g in range(_GROWS):
                pltpu.async_copy(cb_hbm.at[idx_v.at[g]],
                                 rows_v.at[pl.ds(g * GCHUNK, GCHUNK)], sem).wait()
            pltpu.sync_copy(rows_v, out_hbm.at[pl.ds(wid * CAND_W, CAND_W)])

        _SC_GATHER = _sc_gather
    return _SC_GATHER


def _gather_rows(code_book_padded, idx):
    return _build_sc_gather()(code_book_padded,
                              idx.reshape(CAND // GCHUNK, GCHUNK))


def _sum_sq_tree(sq):
    while len(sq) > 1:
        h = len(sq) // 2
        sq = [sq[i] + sq[i + h] for i in range(h)]
    return sq[0]


def _dist_exact(t, rows):
    sq = []
    for c in range(C):
        dlt = t[:, c:c + 1] - rows[:, c:c + 1]
        sq.append(dlt * dlt)
    return jnp.sqrt(_sum_sq_tree(sq))


def _rerank_body(t_ref, r1_ref, r2_ref, i1_ref, i2_ref, out_ref):
    t = t_ref[...]
    d1 = _dist_exact(t, r1_ref[...])[:, 0]
    d2 = _dist_exact(t, r2_ref[...])[:, 0]
    i1 = i1_ref[0, :]
    i2 = i2_ref[0, :]
    pick2 = (d2 < d1) | ((d2 == d1) & (i2 < i1))
    out_ref[0, :] = jnp.where(pick2, i2, i1)


def _tc_rerank(t, rows1, rows2, i1, i2):
    return pl.pallas_call(
        _rerank_body,
        out_shape=jax.ShapeDtypeStruct((1, B * L), jnp.int32),
    )(t, rows1, rows2, i1[None, :], i2[None, :])


def kernel(input_values, proj_weight, code_book):
    x = input_values.reshape(B * L, D)
    cbp = jnp.pad(code_book, ((0, 0), (0, CP - C)))
    t, i1, i2 = _tc_scan(x, proj_weight, code_book.T)
    i1 = i1.reshape(B * L)
    i2 = i2.reshape(B * L)
    idx = jnp.concatenate([i1, i2])
    rows = _gather_rows(cbp, idx)
    labels = _tc_rerank(t, rows[:B * L], rows[B * L:], i1, i2)
    return labels.reshape(B, L)
